# baseline (speedup 1.0000x reference)
import functools

import jax
import jax.numpy as jnp
from jax import lax
from jax.experimental import pallas as pl
from jax.experimental.pallas import tpu as pltpu
from jax.experimental.pallas import tpu_sc as plsc

N = 10000
E = 160000
D_IN = 128
D_MP = 512
K = 10
DELTA = 0.311
EPS = 1e-3
TOTVAR = 0.785
BALANCE = 0.514
QIDX = int(N // K) + 1

BN = 1000
NB = N // BN


def _elu(v):
    return jnp.where(v > 0, v, jnp.exp(jnp.minimum(v, 0.0)) - 1.0)


def _mm1_body(x_ref, w_ref, b_ref, xw_ref, xwc_ref):
    acc = jnp.dot(x_ref[...], w_ref[...], preferred_element_type=jnp.float32)
    acc = acc + b_ref[...]
    xw_ref[...] = acc
    for c in range(4):
        xwc_ref[c] = acc[:, c * 128:(c + 1) * 128]


def _mm1(x, W1, b1):
    return pl.pallas_call(
        _mm1_body,
        grid=(NB,),
        in_specs=[
            pl.BlockSpec((BN, D_IN), lambda i: (i, 0)),
            pl.BlockSpec((D_IN, D_MP), lambda i: (0, 0)),
            pl.BlockSpec((1, D_MP), lambda i: (0, 0)),
        ],
        out_specs=[
            pl.BlockSpec((BN, D_MP), lambda i: (i, 0)),
            pl.BlockSpec((4, BN, 128), lambda i: (0, i, 0)),
        ],
        out_shape=[
            jax.ShapeDtypeStruct((N, D_MP), jnp.float32),
            jax.ShapeDtypeStruct((4, N, 128), jnp.float32),
        ],
    )(x, W1, b1.reshape(1, D_MP))


def _combine_mm_body(xw_ref, degp_ref, aggc_ref, w_ref, b_ref,
                     xw2_ref, xw2c_ref):
    xw = xw_ref[...]
    deg = degp_ref[:, 0] + degp_ref[:, 1]
    agg = jnp.concatenate([aggc_ref[c] for c in range(4)], axis=1)
    h = _elu(xw - DELTA * deg[:, None] * xw + DELTA * agg)
    acc = jnp.dot(h, w_ref[...], preferred_element_type=jnp.float32)
    acc = acc + b_ref[...]
    xw2_ref[...] = acc
    for c in range(4):
        xw2c_ref[c] = acc[:, c * 128:(c + 1) * 128]


def _combine_mm(xw, degp, aggc, W2, b2):
    return pl.pallas_call(
        _combine_mm_body,
        grid=(NB,),
        in_specs=[
            pl.BlockSpec((BN, D_MP), lambda i: (i, 0)),
            pl.BlockSpec((BN, 2), lambda i: (i, 0)),
            pl.BlockSpec((4, BN, 128), lambda i: (0, i, 0)),
            pl.BlockSpec((D_MP, D_MP), lambda i: (0, 0)),
            pl.BlockSpec((1, D_MP), lambda i: (0, 0)),
        ],
        out_specs=[
            pl.BlockSpec((BN, D_MP), lambda i: (i, 0)),
            pl.BlockSpec((4, BN, 128), lambda i: (0, i, 0)),
        ],
        out_shape=[
            jax.ShapeDtypeStruct((N, D_MP), jnp.float32),
            jax.ShapeDtypeStruct((4, N, 128), jnp.float32),
        ],
    )(xw, degp, aggc, W2, b2.reshape(1, D_MP))


def _final_body(xw_ref, degp_ref, aggc_ref, wp_ref, bp_ref, s_ref, s128_ref):
    xw = xw_ref[...]
    deg = degp_ref[:, 0] + degp_ref[:, 1]
    agg = jnp.concatenate([aggc_ref[c] for c in range(4)], axis=1)
    h = _elu(xw - DELTA * deg[:, None] * xw + DELTA * agg)
    logits = jnp.dot(h, wp_ref[...], preferred_element_type=jnp.float32)
    logits = logits + bp_ref[...]
    m = jnp.max(logits, axis=1, keepdims=True)
    e = jnp.exp(logits - m)
    sval = e / jnp.sum(e, axis=1, keepdims=True)
    s_ref[...] = sval
    s128_ref[...] = jnp.concatenate(
        [sval, jnp.zeros((BN, 128 - 16), jnp.float32)], axis=1)


def _final(xw2, degp, aggc, Wp, bp):
    wp_pad = jnp.concatenate([Wp, jnp.zeros((D_MP, 16 - K), jnp.float32)], axis=1)
    bp_pad = jnp.concatenate([bp, jnp.full((16 - K,), -1e30, jnp.float32)])
    return pl.pallas_call(
        _final_body,
        grid=(NB,),
        in_specs=[
            pl.BlockSpec((BN, D_MP), lambda i: (i, 0)),
            pl.BlockSpec((BN, 2), lambda i: (i, 0)),
            pl.BlockSpec((4, BN, 128), lambda i: (0, i, 0)),
            pl.BlockSpec((D_MP, 16), lambda i: (0, 0)),
            pl.BlockSpec((1, 16), lambda i: (0, 0)),
        ],
        out_specs=[
            pl.BlockSpec((BN, 16), lambda i: (i, 0)),
            pl.BlockSpec((BN, 128), lambda i: (i, 0)),
        ],
        out_shape=[
            jax.ShapeDtypeStruct((N, 16), jnp.float32),
            jax.ShapeDtypeStruct((N, 128), jnp.float32),
        ],
    )(xw2, degp, aggc, wp_pad, bp_pad.reshape(1, 16))


def _loss_body(s_ref, tvp_ref, tv_ref, bal_ref):
    tv = jnp.sum(tvp_ref[...]) / (2.0 * E)
    tv_ref[0, 0] = TOTVAR * tv

    s = s_ref[...]
    bits = lax.bitcast_convert_type(s, jnp.int32)
    lo0 = jnp.zeros((1, 16), jnp.int32)
    hi0 = jnp.full((1, 16), 0x7F800000, jnp.int32)

    def body(_, carry):
        lo, hi = carry
        mid = lo + (hi - lo) // 2
        cnt = jnp.sum((bits >= mid).astype(jnp.int32), axis=0, keepdims=True)
        pred = cnt >= QIDX
        return jnp.where(pred, mid, lo), jnp.where(pred, hi, mid)

    lo, hi = lax.fori_loop(0, 31, body, (lo0, hi0))
    med = lax.bitcast_convert_type(lo, jnp.float32)
    diff = s - med
    w = jnp.where(diff >= 0, K - 1.0, 1.0)
    lane = lax.broadcasted_iota(jnp.int32, (1, 16), 1)
    contrib = jnp.where(lane < K, w * jnp.abs(diff), 0.0)
    asym = jnp.sum(contrib)
    denom = N * (K - 1.0)
    bal_ref[0, 0] = BALANCE * ((denom - asym) / denom)


def _losses(s_pad, tvp):
    tv, bal = pl.pallas_call(
        _loss_body,
        grid=(1,),
        in_specs=[
            pl.BlockSpec((N, 16), lambda i: (0, 0)),
            pl.BlockSpec((32, 16), lambda i: (0, 0)),
        ],
        out_specs=[
            pl.BlockSpec((1, 1), lambda i: (0, 0), memory_space=pltpu.SMEM),
            pl.BlockSpec((1, 1), lambda i: (0, 0), memory_space=pltpu.SMEM),
        ],
        out_shape=[
            jax.ShapeDtypeStruct((1, 1), jnp.float32),
            jax.ShapeDtypeStruct((1, 1), jnp.float32),
        ],
    )(s_pad, tvp)
    return tv[0, 0], bal[0, 0]


NC, NS, L = 2, 16, 16
NW = NC * NS
CHA = 64
NCHUNKS = E // CHA
NPAD = 10240
ZB = NPAD // NS

_SC_MESH = plsc.VectorSubcoreMesh(
    core_axis_name="c", subcore_axis_name="s", num_cores=NC, num_subcores=NS)


def _gamma_body(xw_hbm, row_hbm, col_hbm, ew_hbm, gam_hbm, deg_hbm,
                rowv, colv, ewv, rows_r, rows_c, gamv, dbuf, zbuf, deg_sp,
                sem_r, sem_c):
    cid = lax.axis_index("c")
    sid = lax.axis_index("s")
    wid = sid * NC + cid
    for i in range(ZB // L):
        zbuf[pl.ds(i * L, L)] = jnp.zeros((L,), jnp.float32)
    pltpu.sync_copy(zbuf, deg_sp.at[pl.ds(sid * ZB, ZB)])
    plsc.subcore_barrier()

    n_chunks = (NCHUNKS - wid + NW - 1) // NW

    def chunk_body(t, carry):
        off = (wid + t * NW) * CHA
        pltpu.sync_copy(row_hbm.at[pl.ds(off, CHA)], rowv)
        pltpu.sync_copy(col_hbm.at[pl.ds(off, CHA)], colv)
        pltpu.sync_copy(ew_hbm.at[pl.ds(off, CHA)], ewv)
        a = pltpu.async_copy(xw_hbm.at[rowv], rows_r, sem_r)
        b = pltpu.async_copy(xw_hbm.at[colv], rows_c, sem_c)
        a.wait()
        b.wait()

        def edge_body(e, carry2):
            acc = jnp.zeros((L,), jnp.float32)
            for i in range(D_MP // L):
                av = rows_r[e, pl.ds(i * L, L)]
                bv = rows_c[e, pl.ds(i * L, L)]
                acc = acc + jnp.abs(av - bv)
            dbuf[pl.ds(e * (L + 1), L)] = acc
            return carry2

        lax.fori_loop(0, CHA, edge_body, 0)

        lanes = lax.iota(jnp.int32, L)
        for g in range(CHA // L):
            dsum = jnp.zeros((L,), jnp.float32)
            base = g * L * (L + 1)
            for j in range(L):
                dsum = dsum + plsc.load_gather(dbuf, [lanes * (L + 1) + (base + j)])
            ew16 = ewv[pl.ds(g * L, L)]
            gamv[pl.ds(g * L, L)] = ew16 / jnp.maximum(dsum, EPS)
        pltpu.sync_copy(gamv, gam_hbm.at[pl.ds(off, CHA)])
        pltpu.sync_copy(gamv, deg_sp.at[rowv], add=True)
        return carry

    lax.fori_loop(0, n_chunks, chunk_body, 0)

    plsc.subcore_barrier()
    pltpu.sync_copy(deg_sp.at[pl.ds(sid * ZB, ZB)], zbuf)
    pltpu.sync_copy(zbuf, deg_hbm.at[pl.ds(cid * NPAD + sid * ZB, ZB)])


def _sc_gamma(xw, row, col, edge_weight):
    gam, degp = pl.kernel(
        _gamma_body,
        out_type=[
            jax.ShapeDtypeStruct((E,), jnp.float32),
            jax.ShapeDtypeStruct((NC * NPAD,), jnp.float32),
        ],
        mesh=_SC_MESH,
        scratch_types=[
            pltpu.VMEM((CHA,), jnp.int32),
            pltpu.VMEM((CHA,), jnp.int32),
            pltpu.VMEM((CHA,), jnp.float32),
            pltpu.VMEM((CHA, D_MP), jnp.float32),
            pltpu.VMEM((CHA, D_MP), jnp.float32),
            pltpu.VMEM((CHA,), jnp.float32),
            pltpu.VMEM((CHA * (L + 1),), jnp.float32),
            pltpu.VMEM((ZB,), jnp.float32),
            pltpu.VMEM_SHARED((NPAD,), jnp.float32),
            pltpu.SemaphoreType.DMA,
            pltpu.SemaphoreType.DMA,
        ],
        compiler_params=pltpu.CompilerParams(needs_layout_passes=False),
    )(xw, row, col, edge_weight)
    return gam, degp


CHB = 64
NCHB = E // CHB
AGH = NPAD // 2
AGR = AGH + L
SZB = AGR // NS
DZB = AGH // NS


def _agg_body(xwc_hbm, row_hbm, col_hbm, gam_hbm, agg_hbm,
              rowv, rowloc, colv, idxv, gamv, rowsb, dbuf, agg_sp, sem):
    cid = lax.axis_index("c")
    sid = lax.axis_index("s")
    n_chunks = (NCHB - sid + NS - 1) // NS

    def zero_dbuf():
        def zb(i, c):
            for j in range(128 // L):
                dbuf[i, pl.ds(j * L, L)] = jnp.zeros((L,), jnp.float32)
            return c
        lax.fori_loop(0, SZB, zb, 0)

    for q in range(2):
        ck = cid * 2 + q
        for h in range(2):
            zero_dbuf()
            pltpu.sync_copy(dbuf, agg_sp.at[pl.ds(sid * SZB, SZB)])
            plsc.subcore_barrier()

            def chunk_body(t, carry):
                off = (sid + t * NS) * CHB
                pltpu.sync_copy(row_hbm.at[pl.ds(off, CHB)], rowv)
                pltpu.sync_copy(col_hbm.at[pl.ds(off, CHB)], colv)
                pltpu.sync_copy(gam_hbm.at[pl.ds(off, CHB)], gamv)
                for i in range(CHB // L):
                    idxv[pl.ds(i * L, L)] = colv[pl.ds(i * L, L)] + ck * N
                    loc = rowv[pl.ds(i * L, L)] - h * AGH
                    ok = (loc >= 0) & (loc < AGH)
                    rowloc[pl.ds(i * L, L)] = jnp.where(ok, loc, AGH)
                pltpu.async_copy(xwc_hbm.at[idxv], rowsb, sem).wait()

                def edge_body(e, carry2):
                    g16 = plsc.load_gather(gamv, [jnp.full((L,), e, jnp.int32)])
                    for j in range(128 // L):
                        rowsb[e, pl.ds(j * L, L)] = rowsb[e, pl.ds(j * L, L)] * g16
                    return carry2

                lax.fori_loop(0, CHB, edge_body, 0)
                pltpu.sync_copy(rowsb, agg_sp.at[rowloc], add=True)
                return carry

            lax.fori_loop(0, n_chunks, chunk_body, 0)
            plsc.subcore_barrier()
            pltpu.sync_copy(agg_sp.at[pl.ds(sid * DZB, DZB)], dbuf.at[pl.ds(0, DZB)])
            pltpu.sync_copy(dbuf.at[pl.ds(0, DZB)],
                            agg_hbm.at[pl.ds(ck * NPAD + h * AGH + sid * DZB, DZB)])
            plsc.subcore_barrier()


def _sc_agg(xwc, row, col, gamma):
    agg = pl.kernel(
        _agg_body,
        out_type=jax.ShapeDtypeStruct((4 * NPAD, 128), jnp.float32),
        mesh=_SC_MESH,
        scratch_types=[
            pltpu.VMEM((CHB,), jnp.int32),
            pltpu.VMEM((CHB,), jnp.int32),
            pltpu.VMEM((CHB,), jnp.int32),
            pltpu.VMEM((CHB,), jnp.int32),
            pltpu.VMEM((CHB,), jnp.float32),
            pltpu.VMEM((CHB, 128), jnp.float32),
            pltpu.VMEM((SZB, 128), jnp.float32),
            pltpu.VMEM_SHARED((AGR, 128), jnp.float32),
            pltpu.SemaphoreType.DMA,
        ],
        compiler_params=pltpu.CompilerParams(needs_layout_passes=False),
    )(xwc.reshape(4 * N, 128), row, col, gamma)
    return agg.reshape(4, NPAD, 128)[:, :N, :]


CHT = 64


def _tv_body(s_hbm, row_hbm, col_hbm, ew_hbm, tvp_hbm,
             rowv, colv, ewv, rows_r, rows_c, accbuf, sem_r, sem_c):
    cid = lax.axis_index("c")
    sid = lax.axis_index("s")
    wid = sid * NC + cid
    n_chunks = (NCHUNKS - wid + NW - 1) // NW

    def chunk_body(t, acc):
        off = (wid + t * NW) * CHT
        pltpu.sync_copy(row_hbm.at[pl.ds(off, CHT)], rowv)
        pltpu.sync_copy(col_hbm.at[pl.ds(off, CHT)], colv)
        pltpu.sync_copy(ew_hbm.at[pl.ds(off, CHT)], ewv)
        a = pltpu.async_copy(s_hbm.at[rowv], rows_r, sem_r)
        b = pltpu.async_copy(s_hbm.at[colv], rows_c, sem_c)
        a.wait()
        b.wait()

        def edge_body(e, acc2):
            w16 = plsc.load_gather(ewv, [jnp.full((L,), e, jnp.int32)])
            av = rows_r[e, pl.ds(0, L)]
            bv = rows_c[e, pl.ds(0, L)]
            return acc2 + w16 * jnp.abs(av - bv)

        return lax.fori_loop(0, CHT, edge_body, acc)

    acc = lax.fori_loop(0, n_chunks, chunk_body, jnp.zeros((L,), jnp.float32))
    accbuf[...] = acc
    pltpu.sync_copy(accbuf, tvp_hbm.at[pl.ds(wid * L, L)])


def _sc_tv(s128, row, col, edge_weight):
    tvp = pl.kernel(
        _tv_body,
        out_type=jax.ShapeDtypeStruct((NW * L,), jnp.float32),
        mesh=_SC_MESH,
        scratch_types=[
            pltpu.VMEM((CHT,), jnp.int32),
            pltpu.VMEM((CHT,), jnp.int32),
            pltpu.VMEM((CHT,), jnp.float32),
            pltpu.VMEM((CHT, 128), jnp.float32),
            pltpu.VMEM((CHT, 128), jnp.float32),
            pltpu.VMEM((L,), jnp.float32),
            pltpu.SemaphoreType.DMA,
            pltpu.SemaphoreType.DMA,
        ],
        compiler_params=pltpu.CompilerParams(needs_layout_passes=False),
    )(s128, row, col, edge_weight)
    return tvp.reshape(NW, L)


def _edge_stage(xw, xwc, row, col, edge_weight):
    gamma, degflat = _sc_gamma(xw, row, col, edge_weight)
    degp = degflat.reshape(NC, NPAD)[:, :N].T
    aggc = _sc_agg(xwc, row, col, gamma)
    return degp, aggc


def kernel(x, edge_index, edge_weight, W1, b1, W2, b2, Wp, bp):
    row, col = edge_index[0], edge_index[1]
    xw1, xw1c = _mm1(x, W1, b1)
    degp1, aggc1 = _edge_stage(xw1, xw1c, row, col, edge_weight)
    xw2, xw2c = _combine_mm(xw1, degp1, aggc1, W2, b2)
    degp2, aggc2 = _edge_stage(xw2, xw2c, row, col, edge_weight)
    s_pad, s128 = _final(xw2, degp2, aggc2, Wp, bp)
    tvp = _sc_tv(s128, row, col, edge_weight)
    tv_loss, bal_loss = _losses(s_pad, tvp)
    s = s_pad[:, :K]
    return s, tv_loss, bal_loss


# speedup vs baseline: 1.1023x; 1.1023x over previous
import functools

import jax
import jax.numpy as jnp
from jax import lax
from jax.experimental import pallas as pl
from jax.experimental.pallas import tpu as pltpu
from jax.experimental.pallas import tpu_sc as plsc

N = 10000
E = 160000
D_IN = 128
D_MP = 512
K = 10
DELTA = 0.311
EPS = 1e-3
TOTVAR = 0.785
BALANCE = 0.514
QIDX = int(N // K) + 1

BN = 1000
NB = N // BN


def _elu(v):
    return jnp.where(v > 0, v, jnp.exp(jnp.minimum(v, 0.0)) - 1.0)


def _mm1_body(x_ref, w_ref, b_ref, xw_ref, xwc_ref):
    acc = jnp.dot(x_ref[...], w_ref[...], preferred_element_type=jnp.float32)
    acc = acc + b_ref[...]
    xw_ref[...] = acc
    for c in range(4):
        xwc_ref[c] = acc[:, c * 128:(c + 1) * 128]


def _mm1(x, W1, b1):
    return pl.pallas_call(
        _mm1_body,
        grid=(NB,),
        in_specs=[
            pl.BlockSpec((BN, D_IN), lambda i: (i, 0)),
            pl.BlockSpec((D_IN, D_MP), lambda i: (0, 0)),
            pl.BlockSpec((1, D_MP), lambda i: (0, 0)),
        ],
        out_specs=[
            pl.BlockSpec((BN, D_MP), lambda i: (i, 0)),
            pl.BlockSpec((4, BN, 128), lambda i: (0, i, 0)),
        ],
        out_shape=[
            jax.ShapeDtypeStruct((N, D_MP), jnp.float32),
            jax.ShapeDtypeStruct((4, N, 128), jnp.float32),
        ],
    )(x, W1, b1.reshape(1, D_MP))


def _combine_mm_body(xw_ref, degp_ref, aggc_ref, w_ref, b_ref,
                     xw2_ref, xw2c_ref):
    xw = xw_ref[...]
    deg = degp_ref[:, 0] + degp_ref[:, 1]
    agg = jnp.concatenate([aggc_ref[c] for c in range(4)], axis=1)
    h = _elu(xw - DELTA * deg[:, None] * xw + DELTA * agg)
    acc = jnp.dot(h, w_ref[...], preferred_element_type=jnp.float32)
    acc = acc + b_ref[...]
    xw2_ref[...] = acc
    for c in range(4):
        xw2c_ref[c] = acc[:, c * 128:(c + 1) * 128]


def _combine_mm(xw, degp, aggc, W2, b2):
    return pl.pallas_call(
        _combine_mm_body,
        grid=(NB,),
        in_specs=[
            pl.BlockSpec((BN, D_MP), lambda i: (i, 0)),
            pl.BlockSpec((BN, 2), lambda i: (i, 0)),
            pl.BlockSpec((4, BN, 128), lambda i: (0, i, 0)),
            pl.BlockSpec((D_MP, D_MP), lambda i: (0, 0)),
            pl.BlockSpec((1, D_MP), lambda i: (0, 0)),
        ],
        out_specs=[
            pl.BlockSpec((BN, D_MP), lambda i: (i, 0)),
            pl.BlockSpec((4, BN, 128), lambda i: (0, i, 0)),
        ],
        out_shape=[
            jax.ShapeDtypeStruct((N, D_MP), jnp.float32),
            jax.ShapeDtypeStruct((4, N, 128), jnp.float32),
        ],
    )(xw, degp, aggc, W2, b2.reshape(1, D_MP))


def _final_body(xw_ref, degp_ref, aggc_ref, wp_ref, bp_ref, s_ref, s128_ref):
    xw = xw_ref[...]
    deg = degp_ref[:, 0] + degp_ref[:, 1]
    agg = jnp.concatenate([aggc_ref[c] for c in range(4)], axis=1)
    h = _elu(xw - DELTA * deg[:, None] * xw + DELTA * agg)
    logits = jnp.dot(h, wp_ref[...], preferred_element_type=jnp.float32)
    logits = logits + bp_ref[...]
    m = jnp.max(logits, axis=1, keepdims=True)
    e = jnp.exp(logits - m)
    sval = e / jnp.sum(e, axis=1, keepdims=True)
    s_ref[...] = sval
    s128_ref[...] = jnp.concatenate(
        [sval, jnp.zeros((BN, 128 - 16), jnp.float32)], axis=1)


def _final(xw2, degp, aggc, Wp, bp):
    wp_pad = jnp.concatenate([Wp, jnp.zeros((D_MP, 16 - K), jnp.float32)], axis=1)
    bp_pad = jnp.concatenate([bp, jnp.full((16 - K,), -1e30, jnp.float32)])
    return pl.pallas_call(
        _final_body,
        grid=(NB,),
        in_specs=[
            pl.BlockSpec((BN, D_MP), lambda i: (i, 0)),
            pl.BlockSpec((BN, 2), lambda i: (i, 0)),
            pl.BlockSpec((4, BN, 128), lambda i: (0, i, 0)),
            pl.BlockSpec((D_MP, 16), lambda i: (0, 0)),
            pl.BlockSpec((1, 16), lambda i: (0, 0)),
        ],
        out_specs=[
            pl.BlockSpec((BN, 16), lambda i: (i, 0)),
            pl.BlockSpec((BN, 128), lambda i: (i, 0)),
        ],
        out_shape=[
            jax.ShapeDtypeStruct((N, 16), jnp.float32),
            jax.ShapeDtypeStruct((N, 128), jnp.float32),
        ],
    )(xw2, degp, aggc, wp_pad, bp_pad.reshape(1, 16))


def _loss_body(s_ref, tvp_ref, tv_ref, bal_ref):
    tv = jnp.sum(tvp_ref[...]) / (2.0 * E)
    tv_ref[0, 0] = TOTVAR * tv

    s = s_ref[...]
    bits = lax.bitcast_convert_type(s, jnp.int32)
    lo0 = jnp.zeros((1, 16), jnp.int32)
    hi0 = jnp.full((1, 16), 0x7F800000, jnp.int32)

    def body(_, carry):
        lo, hi = carry
        mid = lo + (hi - lo) // 2
        cnt = jnp.sum((bits >= mid).astype(jnp.int32), axis=0, keepdims=True)
        pred = cnt >= QIDX
        return jnp.where(pred, mid, lo), jnp.where(pred, hi, mid)

    lo, hi = lax.fori_loop(0, 31, body, (lo0, hi0))
    med = lax.bitcast_convert_type(lo, jnp.float32)
    diff = s - med
    w = jnp.where(diff >= 0, K - 1.0, 1.0)
    lane = lax.broadcasted_iota(jnp.int32, (1, 16), 1)
    contrib = jnp.where(lane < K, w * jnp.abs(diff), 0.0)
    asym = jnp.sum(contrib)
    denom = N * (K - 1.0)
    bal_ref[0, 0] = BALANCE * ((denom - asym) / denom)


def _losses(s_pad, tvp):
    tv, bal = pl.pallas_call(
        _loss_body,
        grid=(1,),
        in_specs=[
            pl.BlockSpec((N, 16), lambda i: (0, 0)),
            pl.BlockSpec((32, 16), lambda i: (0, 0)),
        ],
        out_specs=[
            pl.BlockSpec((1, 1), lambda i: (0, 0), memory_space=pltpu.SMEM),
            pl.BlockSpec((1, 1), lambda i: (0, 0), memory_space=pltpu.SMEM),
        ],
        out_shape=[
            jax.ShapeDtypeStruct((1, 1), jnp.float32),
            jax.ShapeDtypeStruct((1, 1), jnp.float32),
        ],
    )(s_pad, tvp)
    return tv[0, 0], bal[0, 0]


NC, NS, L = 2, 16, 16
NW = NC * NS
E_PAD = 163840
NPAD = 10240
ZB = NPAD // NS

CHA = 32
NT_A = E_PAD // CHA // NW

_SC_MESH = plsc.VectorSubcoreMesh(
    core_axis_name="c", subcore_axis_name="s", num_cores=NC, num_subcores=NS)
_SC_PARAMS = pltpu.CompilerParams(needs_layout_passes=False)


def _gamma_body(xw_hbm, row_hbm, col_hbm, ew_hbm, gam_hbm, deg_hbm,
                rowv0, rowv1, colv0, colv1, ewv0, ewv1,
                rr0, rr1, rc0, rc1, gamv, dbuf, zbuf, deg_sp,
                semr0, semr1, semc0, semc1):
    cid = lax.axis_index("c")
    sid = lax.axis_index("s")
    wid = sid * NC + cid
    rowv, colv, ewv = [rowv0, rowv1], [colv0, colv1], [ewv0, ewv1]
    rr, rc = [rr0, rr1], [rc0, rc1]
    semr, semc = [semr0, semr1], [semc0, semc1]

    for i in range(ZB // L):
        zbuf[pl.ds(i * L, L)] = jnp.zeros((L,), jnp.float32)
    pltpu.sync_copy(zbuf, deg_sp.at[pl.ds(sid * ZB, ZB)])
    plsc.subcore_barrier()

    def fetch(t, b):
        off = (wid + t * NW) * CHA
        pltpu.sync_copy(row_hbm.at[pl.ds(off, CHA)], rowv[b])
        pltpu.sync_copy(col_hbm.at[pl.ds(off, CHA)], colv[b])
        pltpu.sync_copy(ew_hbm.at[pl.ds(off, CHA)], ewv[b])
        pltpu.async_copy(xw_hbm.at[rowv[b]], rr[b], semr[b])
        pltpu.async_copy(xw_hbm.at[colv[b]], rc[b], semc[b])

    fetch(0, 0)
    lanes = lax.iota(jnp.int32, L)

    def outer(it, carry):
        for b in range(2):
            t = it * 2 + b
            pltpu.make_async_copy(xw_hbm.at[rowv[b]], rr[b], semr[b]).wait()
            pltpu.make_async_copy(xw_hbm.at[colv[b]], rc[b], semc[b]).wait()

            @pl.when(t + 1 < NT_A)
            def _():
                fetch(t + 1, 1 - b)

            @plsc.parallel_loop(0, CHA, unroll=2)
            def _edges(e):
                acc = jnp.zeros((L,), jnp.float32)
                for i in range(D_MP // L):
                    acc = acc + jnp.abs(rr[b][e, pl.ds(i * L, L)]
                                        - rc[b][e, pl.ds(i * L, L)])
                dbuf[pl.ds(e * (L + 1), L)] = acc

            for g in range(CHA // L):
                dsum = jnp.zeros((L,), jnp.float32)
                base = g * L * (L + 1)
                for j in range(L):
                    dsum = dsum + plsc.load_gather(
                        dbuf, [lanes * (L + 1) + (base + j)])
                ew16 = ewv[b][pl.ds(g * L, L)]
                gamv[pl.ds(g * L, L)] = ew16 / jnp.maximum(dsum, EPS)

            off = (wid + t * NW) * CHA
            pltpu.sync_copy(gamv, gam_hbm.at[pl.ds(off, CHA)])
            pltpu.sync_copy(gamv, deg_sp.at[rowv[b]], add=True)
        return carry

    lax.fori_loop(0, NT_A // 2, outer, 0)

    plsc.subcore_barrier()
    pltpu.sync_copy(deg_sp.at[pl.ds(sid * ZB, ZB)], zbuf)
    pltpu.sync_copy(zbuf, deg_hbm.at[pl.ds(cid * NPAD + sid * ZB, ZB)])


def _sc_gamma(xw, row, col, edge_weight):
    gam, degp = pl.kernel(
        _gamma_body,
        out_type=[
            jax.ShapeDtypeStruct((E_PAD,), jnp.float32),
            jax.ShapeDtypeStruct((NC * NPAD,), jnp.float32),
        ],
        mesh=_SC_MESH,
        scratch_types=[
            pltpu.VMEM((CHA,), jnp.int32),
            pltpu.VMEM((CHA,), jnp.int32),
            pltpu.VMEM((CHA,), jnp.int32),
            pltpu.VMEM((CHA,), jnp.int32),
            pltpu.VMEM((CHA,), jnp.float32),
            pltpu.VMEM((CHA,), jnp.float32),
            pltpu.VMEM((CHA, D_MP), jnp.float32),
            pltpu.VMEM((CHA, D_MP), jnp.float32),
            pltpu.VMEM((CHA, D_MP), jnp.float32),
            pltpu.VMEM((CHA, D_MP), jnp.float32),
            pltpu.VMEM((CHA,), jnp.float32),
            pltpu.VMEM((CHA * (L + 1),), jnp.float32),
            pltpu.VMEM((ZB,), jnp.float32),
            pltpu.VMEM_SHARED((NPAD,), jnp.float32),
            pltpu.SemaphoreType.DMA,
            pltpu.SemaphoreType.DMA,
            pltpu.SemaphoreType.DMA,
            pltpu.SemaphoreType.DMA,
        ],
        compiler_params=_SC_PARAMS,
    )(xw, row, col, edge_weight)
    return gam, degp


CHB = 128
NT_B = E_PAD // CHB // NS
AGH = NPAD // 2
AGR = AGH + L
SZB = AGR // NS
DZB = AGH // NS


def _agg_body(xwc_hbm, row_hbm, col_hbm, gam_hbm, agg_hbm,
              rl0, rl1, colv0, colv1, idx0, idx1, gv0, gv1,
              rb0, rb1, dbuf, agg_sp, semg0, semg1, sems0, sems1):
    cid = lax.axis_index("c")
    sid = lax.axis_index("s")
    rl, colv, idxv, gv = [rl0, rl1], [colv0, colv1], [idx0, idx1], [gv0, gv1]
    rb, semg, sems = [rb0, rb1], [semg0, semg1], [sems0, sems1]

    def zero_dbuf():
        def zb(i, c):
            for j in range(128 // L):
                dbuf[i, pl.ds(j * L, L)] = jnp.zeros((L,), jnp.float32)
            return c
        lax.fori_loop(0, SZB, zb, 0)

    def fetch(t, b, ck, h):
        @pl.when(t >= 2)
        def _():
            pltpu.make_async_copy(rb[b], agg_sp.at[rl[b]], sems[b]).wait()
        off = (sid + t * NS) * CHB
        pltpu.sync_copy(row_hbm.at[pl.ds(off, CHB)], rl[b])
        pltpu.sync_copy(col_hbm.at[pl.ds(off, CHB)], colv[b])
        pltpu.sync_copy(gam_hbm.at[pl.ds(off, CHB)], gv[b])
        for i in range(CHB // L):
            idxv[b][pl.ds(i * L, L)] = colv[b][pl.ds(i * L, L)] + ck * N
            loc = rl[b][pl.ds(i * L, L)] - h * AGH
            ok = (loc >= 0) & (loc < AGH)
            rl[b][pl.ds(i * L, L)] = jnp.where(ok, loc, AGH)
        pltpu.async_copy(xwc_hbm.at[idxv[b]], rb[b], semg[b])

    for q in range(2):
        ck = cid * 2 + q
        for h in range(2):
            zero_dbuf()
            pltpu.sync_copy(dbuf, agg_sp.at[pl.ds(sid * SZB, SZB)])
            plsc.subcore_barrier()
            fetch(0, 0, ck, h)

            def outer(it, carry):
                for b in range(2):
                    t = it * 2 + b
                    pltpu.make_async_copy(
                        xwc_hbm.at[idxv[b]], rb[b], semg[b]).wait()

                    @pl.when(t + 1 < NT_B)
                    def _():
                        fetch(t + 1, 1 - b, ck, h)

                    @plsc.parallel_loop(0, CHB, unroll=2)
                    def _edges(e):
                        g16 = plsc.load_gather(
                            gv[b], [jnp.full((L,), e, jnp.int32)])
                        for j in range(128 // L):
                            rb[b][e, pl.ds(j * L, L)] = (
                                rb[b][e, pl.ds(j * L, L)] * g16)

                    pltpu.async_copy(rb[b], agg_sp.at[rl[b]], sems[b],
                                     add=True)
                return carry

            lax.fori_loop(0, NT_B // 2, outer, 0)
            for b in range(2):
                pltpu.make_async_copy(rb[b], agg_sp.at[rl[b]], sems[b]).wait()
            plsc.subcore_barrier()
            pltpu.sync_copy(agg_sp.at[pl.ds(sid * DZB, DZB)],
                            dbuf.at[pl.ds(0, DZB)])
            pltpu.sync_copy(dbuf.at[pl.ds(0, DZB)],
                            agg_hbm.at[pl.ds(ck * NPAD + h * AGH + sid * DZB,
                                             DZB)])
            plsc.subcore_barrier()


def _sc_agg(xwc, row, col, gamma):
    agg = pl.kernel(
        _agg_body,
        out_type=jax.ShapeDtypeStruct((4 * NPAD, 128), jnp.float32),
        mesh=_SC_MESH,
        scratch_types=[
            pltpu.VMEM((CHB,), jnp.int32),
            pltpu.VMEM((CHB,), jnp.int32),
            pltpu.VMEM((CHB,), jnp.int32),
            pltpu.VMEM((CHB,), jnp.int32),
            pltpu.VMEM((CHB,), jnp.int32),
            pltpu.VMEM((CHB,), jnp.int32),
            pltpu.VMEM((CHB,), jnp.float32),
            pltpu.VMEM((CHB,), jnp.float32),
            pltpu.VMEM((CHB, 128), jnp.float32),
            pltpu.VMEM((CHB, 128), jnp.float32),
            pltpu.VMEM((SZB, 128), jnp.float32),
            pltpu.VMEM_SHARED((AGR, 128), jnp.float32),
            pltpu.SemaphoreType.DMA,
            pltpu.SemaphoreType.DMA,
            pltpu.SemaphoreType.DMA,
            pltpu.SemaphoreType.DMA,
        ],
        compiler_params=_SC_PARAMS,
    )(xwc.reshape(4 * N, 128), row, col, gamma)
    return agg.reshape(4, NPAD, 128)[:, :N, :]


CHT = 128
NT_T = E_PAD // CHT // NW


def _tv_body(s_hbm, row_hbm, col_hbm, ew_hbm, tvp_hbm,
             rowv0, rowv1, colv0, colv1, ewv0, ewv1,
             rr0, rr1, rc0, rc1, accbuf, semr0, semr1, semc0, semc1):
    cid = lax.axis_index("c")
    sid = lax.axis_index("s")
    wid = sid * NC + cid
    rowv, colv, ewv = [rowv0, rowv1], [colv0, colv1], [ewv0, ewv1]
    rr, rc = [rr0, rr1], [rc0, rc1]
    semr, semc = [semr0, semr1], [semc0, semc1]

    def fetch(t, b):
        off = (wid + t * NW) * CHT
        pltpu.sync_copy(row_hbm.at[pl.ds(off, CHT)], rowv[b])
        pltpu.sync_copy(col_hbm.at[pl.ds(off, CHT)], colv[b])
        pltpu.sync_copy(ew_hbm.at[pl.ds(off, CHT)], ewv[b])
        pltpu.async_copy(s_hbm.at[rowv[b]], rr[b], semr[b])
        pltpu.async_copy(s_hbm.at[colv[b]], rc[b], semc[b])

    fetch(0, 0)

    def outer(it, acc):
        for b in range(2):
            t = it * 2 + b
            pltpu.make_async_copy(s_hbm.at[rowv[b]], rr[b], semr[b]).wait()
            pltpu.make_async_copy(s_hbm.at[colv[b]], rc[b], semc[b]).wait()

            @pl.when(t + 1 < NT_T)
            def _():
                fetch(t + 1, 1 - b)

            def _edges(e, a):
                w16 = plsc.load_gather(ewv[b], [jnp.full((L,), e, jnp.int32)])
                av = rr[b][e, pl.ds(0, L)]
                bv = rc[b][e, pl.ds(0, L)]
                return a + w16 * jnp.abs(av - bv)

            acc = plsc.parallel_loop(0, CHT, unroll=2, carry=acc)(_edges)
        return acc

    acc = lax.fori_loop(0, NT_T // 2, outer, jnp.zeros((L,), jnp.float32))
    accbuf[...] = acc
    pltpu.sync_copy(accbuf, tvp_hbm.at[pl.ds(wid * L, L)])


def _sc_tv(s128, row, col, edge_weight):
    tvp = pl.kernel(
        _tv_body,
        out_type=jax.ShapeDtypeStruct((NW * L,), jnp.float32),
        mesh=_SC_MESH,
        scratch_types=[
            pltpu.VMEM((CHT,), jnp.int32),
            pltpu.VMEM((CHT,), jnp.int32),
            pltpu.VMEM((CHT,), jnp.int32),
            pltpu.VMEM((CHT,), jnp.int32),
            pltpu.VMEM((CHT,), jnp.float32),
            pltpu.VMEM((CHT,), jnp.float32),
            pltpu.VMEM((CHT, 128), jnp.float32),
            pltpu.VMEM((CHT, 128), jnp.float32),
            pltpu.VMEM((CHT, 128), jnp.float32),
            pltpu.VMEM((CHT, 128), jnp.float32),
            pltpu.VMEM((L,), jnp.float32),
            pltpu.SemaphoreType.DMA,
            pltpu.SemaphoreType.DMA,
            pltpu.SemaphoreType.DMA,
            pltpu.SemaphoreType.DMA,
        ],
        compiler_params=_SC_PARAMS,
    )(s128, row, col, edge_weight)
    return tvp.reshape(NW, L)


def _edge_stage(xw, xwc, row, col, edge_weight):
    gamma, degflat = _sc_gamma(xw, row, col, edge_weight)
    degp = degflat.reshape(NC, NPAD)[:, :N].T
    aggc = _sc_agg(xwc, row, col, gamma)
    return degp, aggc


def kernel(x, edge_index, edge_weight, W1, b1, W2, b2, Wp, bp):
    npad = E_PAD - E
    row = jnp.concatenate([edge_index[0], jnp.zeros((npad,), jnp.int32)])
    col = jnp.concatenate([edge_index[1], jnp.zeros((npad,), jnp.int32)])
    ew = jnp.concatenate([edge_weight, jnp.zeros((npad,), jnp.float32)])
    xw1, xw1c = _mm1(x, W1, b1)
    degp1, aggc1 = _edge_stage(xw1, xw1c, row, col, ew)
    xw2, xw2c = _combine_mm(xw1, degp1, aggc1, W2, b2)
    degp2, aggc2 = _edge_stage(xw2, xw2c, row, col, ew)
    s_pad, s128 = _final(xw2, degp2, aggc2, Wp, bp)
    tvp = _sc_tv(s128, row, col, ew)
    tv_loss, bal_loss = _losses(s_pad, tvp)
    s = s_pad[:, :K]
    return s, tv_loss, bal_loss
